# Initial kernel scaffold; baseline (speedup 1.0000x reference)
import jax
import jax.numpy as jnp
from jax.experimental import pallas as pl


def kernel(features, centres, centre_labels, weight):
    raise NotImplementedError("write your pallas kernel here")


# speedup vs baseline: 37.1890x; 37.1890x over previous
import functools

import jax
import jax.numpy as jnp
from jax import lax
from jax.experimental import pallas as pl
from jax.experimental.pallas import tpu as pltpu
from jax.experimental.pallas import tpu_sc as plsc

B = 1024
D = 128
M = 100000
MTILE = 1024
NMT = 98
MP = MTILE * NMT
K = 64
NCLS = 1000
CPAD = 1008
GC = 0.5
NCHUNK = 8
CH = MP // NCHUNK
NGRP = CH // 128
NWORK = 32
ROWS_PER = B // NWORK
CAP = 192


def _dist_kernel(f_ref, c_ref, o_ref):
    f = f_ref[...]
    c = c_ref[...]
    dot = lax.dot_general(f, c, (((1,), (1,)), ((), ())),
                          preferred_element_type=jnp.float32)
    x2 = jnp.sum(f * f, axis=1, keepdims=True)
    c2 = jnp.sum(c * c, axis=1)[None, :]
    sq = jnp.maximum(x2 + c2 - 2.0 * dot, 0.0)
    col = lax.broadcasted_iota(jnp.int32, (B, MTILE), 1) + pl.program_id(0) * MTILE
    o_ref[...] = jnp.where(col >= M, jnp.inf, sq)


def _sort16(v, i):
    return plsc.sort_key_val(v, i)


def _minpair(av, ai, bv, bi):
    m = av <= bv
    return jnp.where(m, av, bv), jnp.where(m, ai, bi)


def _maxpair(av, ai, bv, bi):
    m = av <= bv
    return jnp.where(m, bv, av), jnp.where(m, bi, ai)


def _rev(x):
    return lax.rev(x, (0,))


def _merge2(a, b):
    bv, bi = _rev(b[0]), _rev(b[1])
    lv, li = _minpair(a[0], a[1], bv, bi)
    hv, hi = _maxpair(a[0], a[1], bv, bi)
    return _sort16(lv, li), _sort16(hv, hi)


def _sort64(vs, is_):
    s = [_sort16(vs[k], is_[k]) for k in range(4)]
    a0, a1 = _merge2(s[0], s[1])
    a2, a3 = _merge2(s[2], s[3])
    r3 = (_rev(a3[0]), _rev(a3[1]))
    r2 = (_rev(a2[0]), _rev(a2[1]))
    l0 = _minpair(*a0, *r3)
    h0 = _maxpair(*a0, *r3)
    l1 = _minpair(*a1, *r2)
    h1 = _maxpair(*a1, *r2)
    c0 = _minpair(*l0, *l1)
    c1 = _maxpair(*l0, *l1)
    d0 = _minpair(*h0, *h1)
    d1 = _maxpair(*h0, *h1)
    return [_sort16(*c0), _sort16(*c1), _sort16(*d0), _sort16(*d1)]


def _merge_keep64(r, s):
    l = []
    for k in range(4):
        srv, sri = _rev(s[3 - k][0]), _rev(s[3 - k][1])
        l.append(_minpair(r[k][0], r[k][1], srv, sri))
    a0 = _minpair(*l[0], *l[2])
    a2 = _maxpair(*l[0], *l[2])
    a1 = _minpair(*l[1], *l[3])
    a3 = _maxpair(*l[1], *l[3])
    b0 = _minpair(*a0, *a1)
    b1 = _maxpair(*a0, *a1)
    b2 = _minpair(*a2, *a3)
    b3 = _maxpair(*a2, *a3)
    return [_sort16(*b0), _sort16(*b1), _sort16(*b2), _sort16(*b3)]


def _sc_body(sq_hbm, lbl_hbm, w_hbm, out_hbm,
             buf, cand_v, cand_i, r_v, r_i,
             dvals, idxs, lblv, wvals, p_all, t_ref, cnt_ref, sem, sem2):
    wid = lax.axis_index("s") * 2 + lax.axis_index("c")
    iota16 = lax.broadcasted_iota(jnp.int32, (16,), 0)
    inf16 = jnp.full((16,), jnp.inf, jnp.float32)
    zero16i = jnp.zeros((16,), jnp.int32)

    def compact():
        cnt16 = jnp.full((16,), cnt_ref[0], jnp.int32)
        rr_ = [(r_v[pl.ds(16 * k, 16)], r_i[pl.ds(16 * k, 16)]) for k in range(4)]
        for blk in range(3):
            sv = []
            si = []
            for k in range(4):
                pos = jnp.full((16,), blk * 64 + 16 * k, jnp.int32) + iota16
                valid = pos < cnt16
                sv.append(jnp.where(valid, cand_v[pl.ds(blk * 64 + 16 * k, 16)],
                                    inf16))
                si.append(cand_i[pl.ds(blk * 64 + 16 * k, 16)])
            rr_ = _merge_keep64(rr_, _sort64(sv, si))
        for k in range(4):
            r_v[pl.ds(16 * k, 16)] = rr_[k][0]
            r_i[pl.ds(16 * k, 16)] = rr_[k][1]
        t_ref[0] = jnp.max(rr_[3][0])
        cnt_ref[0] = 0

    def scan_row(rr, _):
        row = wid * ROWS_PER + rr
        for k in range(4):
            r_v[pl.ds(16 * k, 16)] = inf16
            r_i[pl.ds(16 * k, 16)] = zero16i
        t_ref[0] = jnp.inf
        cnt_ref[0] = 0

        def chunk_body(c, _):
            pltpu.sync_copy(sq_hbm.at[row, c], buf)

            def grp_body(g, _):
                base = g * 128
                vs = [buf[pl.ds(base + 16 * k, 16)] for k in range(8)]
                m01 = jnp.minimum(vs[0], vs[1])
                m23 = jnp.minimum(vs[2], vs[3])
                m45 = jnp.minimum(vs[4], vs[5])
                m67 = jnp.minimum(vs[6], vs[7])
                mtree = jnp.minimum(jnp.minimum(m01, m23), jnp.minimum(m45, m67))
                mn = jnp.min(mtree)

                @pl.when(mn <= t_ref[0])
                def _():
                    @pl.when(cnt_ref[0] > 64)
                    def _():
                        compact()

                    t16 = jnp.full((16,), t_ref[0], jnp.float32)
                    gbase = c * CH + base
                    for k in range(8):
                        mk = vs[k] <= t16
                        cnt = cnt_ref[0]
                        plsc.store_compressed(cand_v.at[pl.ds(cnt, 16)], vs[k], mask=mk)
                        plsc.store_compressed(
                            cand_i.at[pl.ds(cnt, 16)],
                            jnp.full((16,), gbase + 16 * k, jnp.int32) + iota16,
                            mask=mk)
                        cnt_ref[0] = cnt + jnp.max(
                            plsc.all_reduce_population_count(mk))

                return 0

            lax.fori_loop(0, NGRP, grp_body, 0)
            return 0

        lax.fori_loop(0, NCHUNK, chunk_body, 0)
        compact()
        for k in range(4):
            dvals[pl.ds(rr * K + 16 * k, 16)] = r_v[pl.ds(16 * k, 16)]
            idxs[pl.ds(rr * K + 16 * k, 16)] = r_i[pl.ds(16 * k, 16)]
        return 0

    lax.fori_loop(0, ROWS_PER, scan_row, 0)

    copies = []
    for g in range(ROWS_PER * K // 128):
        sl = pl.ds(g * 128, 128)
        copies.append(pltpu.async_copy(lbl_hbm.at[idxs.at[sl]], lblv.at[sl], sem))
        copies.append(pltpu.async_copy(w_hbm.at[idxs.at[sl]], wvals.at[sl], sem2))
    for cp in copies:
        cp.wait()

    prev_idx = jnp.maximum(iota16 - 1, 0)
    next_idx = jnp.minimum(iota16 + 1, 15)

    def finalize_row(rr, _):
        def zero_body(j, _):
            p_all[pl.ds(rr * CPAD + 16 * j, 16)] = jnp.zeros((16,), jnp.float32)
            return 0

        lax.fori_loop(0, CPAD // 16, zero_body, 0)
        pbase16 = jnp.full((16,), rr * CPAD, jnp.int32)
        for k in range(4):
            off = pl.ds(rr * K + 16 * k, 16)
            e = jnp.exp(wvals[off] - dvals[off] * GC)
            ls, es = plsc.sort_key_val(lblv[off], e)
            cs = plsc.cumsum(es)
            pl_ = jnp.take_along_axis(ls, prev_idx, axis=0,
                                      mode="promise_in_bounds")
            nl_ = jnp.take_along_axis(ls, next_idx, axis=0,
                                      mode="promise_in_bounds")
            is_start = (iota16 == 0) | (ls != pl_)
            is_end = (iota16 == 15) | (ls != nl_)
            pre = jnp.where(iota16 == 0, 0.0,
                            jnp.take_along_axis(cs, prev_idx, axis=0,
                                                mode="promise_in_bounds"))
            plsc.addupdate_scatter(p_all, [pbase16 + ls], cs, mask=is_end)
            plsc.addupdate_scatter(p_all, [pbase16 + ls], -pre, mask=is_start)
        return 0

    lax.fori_loop(0, ROWS_PER, finalize_row, 0)
    pltpu.sync_copy(p_all, out_hbm.at[pl.ds(wid * ROWS_PER * CPAD, ROWS_PER * CPAD)])


_sc_topk = functools.partial(
    pl.kernel,
    out_type=jax.ShapeDtypeStruct((B * CPAD,), jnp.float32),
    mesh=plsc.VectorSubcoreMesh(core_axis_name="c", subcore_axis_name="s"),
    compiler_params=pltpu.CompilerParams(needs_layout_passes=False),
    scratch_types=[
        pltpu.VMEM((CH,), jnp.float32),
        pltpu.VMEM((CAP,), jnp.float32),
        pltpu.VMEM((CAP,), jnp.int32),
        pltpu.VMEM((K,), jnp.float32),
        pltpu.VMEM((K,), jnp.int32),
        pltpu.VMEM((ROWS_PER * K,), jnp.float32),
        pltpu.VMEM((ROWS_PER * K,), jnp.int32),
        pltpu.VMEM((ROWS_PER * K,), jnp.int32),
        pltpu.VMEM((ROWS_PER * K,), jnp.float32),
        pltpu.VMEM((ROWS_PER * CPAD,), jnp.float32),
        pltpu.SMEM((1,), jnp.float32),
        pltpu.SMEM((1,), jnp.int32),
        pltpu.SemaphoreType.DMA,
        pltpu.SemaphoreType.DMA,
    ],
)(_sc_body)


def _norm_kernel(p_ref, o_ref):
    p = p_ref[...]
    p = jnp.where(p == 0.0, 1e-10, p)
    p = p / jnp.sum(p, axis=1, keepdims=True)
    o_ref[...] = jnp.log(p)


def kernel(features, centres, centre_labels, weight):
    centres_p = jnp.concatenate(
        [centres, jnp.zeros((MP - M, D), jnp.float32)], axis=0)
    sq = pl.pallas_call(
        _dist_kernel,
        grid=(NMT,),
        in_specs=[
            pl.BlockSpec((B, D), lambda m: (0, 0)),
            pl.BlockSpec((MTILE, D), lambda m: (m, 0)),
        ],
        out_specs=pl.BlockSpec((B, MTILE), lambda m: (0, m)),
        out_shape=jax.ShapeDtypeStruct((B, MP), jnp.float32),
    )(features, centres_p)
    sq3 = sq.reshape(B, NCHUNK, CH)
    pflat = _sc_topk(sq3, centre_labels, weight)
    p = pflat.reshape(B, CPAD)[:, :NCLS]
    out = pl.pallas_call(
        _norm_kernel,
        out_shape=jax.ShapeDtypeStruct((B, NCLS), jnp.float32),
    )(p)
    return out


# speedup vs baseline: 88.8088x; 2.3880x over previous
import functools

import jax
import jax.numpy as jnp
from jax import lax
from jax.experimental import pallas as pl
from jax.experimental.pallas import tpu as pltpu
from jax.experimental.pallas import tpu_sc as plsc

B = 1024
D = 128
M = 100000
MTILE = 1024
NMT = 98
MP = MTILE * NMT
NG = MP // 128
NGPAD = 1024
K = 64
NCLS = 1000
CPAD = 1008
GC = 0.5
NWORK = 32
ROWS_PER = B // NWORK
CAP = 192


def _dist_kernel(f_ref, c_ref, o_ref, g_ref):
    f = f_ref[...]
    c = c_ref[...]
    dot = lax.dot_general(f, c, (((1,), (1,)), ((), ())),
                          preferred_element_type=jnp.float32)
    x2 = jnp.sum(f * f, axis=1, keepdims=True)
    c2 = jnp.sum(c * c, axis=1)[None, :]
    sq = jnp.maximum(x2 + c2 - 2.0 * dot, 0.0)
    col = lax.broadcasted_iota(jnp.int32, (B, MTILE), 1) + pl.program_id(0) * MTILE
    sq = jnp.where(col >= M, jnp.inf, sq)
    o_ref[...] = sq
    mins = [jnp.min(sq[:, 128 * g:128 * (g + 1)], axis=1, keepdims=True)
            for g in range(8)]
    g_ref[...] = jnp.concatenate(mins, axis=1)[None]


def _sort16(v, i):
    return plsc.sort_key_val(v, i)


def _minpair(av, ai, bv, bi):
    m = av <= bv
    return jnp.where(m, av, bv), jnp.where(m, ai, bi)


def _maxpair(av, ai, bv, bi):
    m = av <= bv
    return jnp.where(m, bv, av), jnp.where(m, bi, ai)


def _rev(x):
    return lax.rev(x, (0,))


def _merge2(a, b):
    bv, bi = _rev(b[0]), _rev(b[1])
    lv, li = _minpair(a[0], a[1], bv, bi)
    hv, hi = _maxpair(a[0], a[1], bv, bi)
    return _sort16(lv, li), _sort16(hv, hi)


def _sort64(vs, is_):
    s = [_sort16(vs[k], is_[k]) for k in range(4)]
    a0, a1 = _merge2(s[0], s[1])
    a2, a3 = _merge2(s[2], s[3])
    r3 = (_rev(a3[0]), _rev(a3[1]))
    r2 = (_rev(a2[0]), _rev(a2[1]))
    l0 = _minpair(*a0, *r3)
    h0 = _maxpair(*a0, *r3)
    l1 = _minpair(*a1, *r2)
    h1 = _maxpair(*a1, *r2)
    c0 = _minpair(*l0, *l1)
    c1 = _maxpair(*l0, *l1)
    d0 = _minpair(*h0, *h1)
    d1 = _maxpair(*h0, *h1)
    return [_sort16(*c0), _sort16(*c1), _sort16(*d0), _sort16(*d1)]


def _merge_keep64(r, s):
    l = []
    for k in range(4):
        srv, sri = _rev(s[3 - k][0]), _rev(s[3 - k][1])
        l.append(_minpair(r[k][0], r[k][1], srv, sri))
    a0 = _minpair(*l[0], *l[2])
    a2 = _maxpair(*l[0], *l[2])
    a1 = _minpair(*l[1], *l[3])
    a3 = _maxpair(*l[1], *l[3])
    b0 = _minpair(*a0, *a1)
    b1 = _maxpair(*a0, *a1)
    b2 = _minpair(*a2, *a3)
    b3 = _maxpair(*a2, *a3)
    return [_sort16(*b0), _sort16(*b1), _sort16(*b2), _sort16(*b3)]


def _sc_body(sqtab_hbm, gmin_hbm, lbl_hbm, w_hbm, out_hbm,
             gbuf, hot, gids, idxdma, cand_v, cand_i, r_v, r_i,
             dvals, idxs, lblv, wvals, p_all, t_ref, cnt_ref, sem, sem2):
    wid = lax.axis_index("s") * 2 + lax.axis_index("c")
    iota16 = lax.broadcasted_iota(jnp.int32, (16,), 0)
    inf16 = jnp.full((16,), jnp.inf, jnp.float32)
    zero16i = jnp.zeros((16,), jnp.int32)

    def compact():
        cnt16 = jnp.full((16,), cnt_ref[0], jnp.int32)
        rr_ = [(r_v[pl.ds(16 * k, 16)], r_i[pl.ds(16 * k, 16)]) for k in range(4)]
        for blk in range(3):
            sv = []
            si = []
            for k in range(4):
                pos = jnp.full((16,), blk * 64 + 16 * k, jnp.int32) + iota16
                valid = pos < cnt16
                sv.append(jnp.where(valid, cand_v[pl.ds(blk * 64 + 16 * k, 16)],
                                    inf16))
                si.append(cand_i[pl.ds(blk * 64 + 16 * k, 16)])
            rr_ = _merge_keep64(rr_, _sort64(sv, si))
        for k in range(4):
            r_v[pl.ds(16 * k, 16)] = rr_[k][0]
            r_i[pl.ds(16 * k, 16)] = rr_[k][1]
        t_ref[0] = jnp.max(rr_[3][0])
        cnt_ref[0] = 0

    def reset_r():
        for k in range(4):
            r_v[pl.ds(16 * k, 16)] = inf16
            r_i[pl.ds(16 * k, 16)] = zero16i
        cnt_ref[0] = 0

    def group_step(vs, idxvecs):
        m01 = jnp.minimum(vs[0], vs[1])
        m23 = jnp.minimum(vs[2], vs[3])
        m45 = jnp.minimum(vs[4], vs[5])
        m67 = jnp.minimum(vs[6], vs[7])
        mtree = jnp.minimum(jnp.minimum(m01, m23), jnp.minimum(m45, m67))
        mn = jnp.min(mtree)

        @pl.when(mn <= t_ref[0])
        def _():
            @pl.when(cnt_ref[0] > 64)
            def _():
                compact()

            t16 = jnp.full((16,), t_ref[0], jnp.float32)
            for k in range(8):
                mk = vs[k] <= t16
                cnt = cnt_ref[0]
                plsc.store_compressed(cand_v.at[pl.ds(cnt, 16)], vs[k], mask=mk)
                plsc.store_compressed(cand_i.at[pl.ds(cnt, 16)], idxvecs[k],
                                      mask=mk)
                cnt_ref[0] = cnt + jnp.max(plsc.all_reduce_population_count(mk))

    def scan_row(rr, _):
        row = wid * ROWS_PER + rr
        pltpu.sync_copy(gmin_hbm.at[row], gbuf)
        reset_r()
        t_ref[0] = jnp.inf

        def mini_group(g, _):
            base = g * 128
            vs = [gbuf[pl.ds(base + 16 * k, 16)] for k in range(8)]
            ivs = [jnp.full((16,), base + 16 * k, jnp.int32) + iota16
                   for k in range(8)]
            group_step(vs, ivs)
            return 0

        lax.fori_loop(0, NGPAD // 128, mini_group, 0)
        compact()

        rowbase = jnp.full((16,), row * NG, jnp.int32)
        for k in range(4):
            gv = r_i[pl.ds(16 * k, 16)]
            gids[pl.ds(16 * k, 16)] = gv
            idxdma[pl.ds(16 * k, 16)] = gv + rowbase
        pltpu.async_copy(sqtab_hbm.at[idxdma], hot, sem).wait()

        reset_r()

        def hot_group(s, _):
            gidv = plsc.load_gather(gids, [jnp.full((16,), s, jnp.int32)])
            base16 = gidv * 128
            vs = [hot[s, pl.ds(16 * k, 16)] for k in range(8)]
            ivs = [base16 + (jnp.full((16,), 16 * k, jnp.int32) + iota16)
                   for k in range(8)]
            group_step(vs, ivs)
            return 0

        lax.fori_loop(0, K, hot_group, 0)
        compact()
        for k in range(4):
            dvals[pl.ds(rr * K + 16 * k, 16)] = r_v[pl.ds(16 * k, 16)]
            idxs[pl.ds(rr * K + 16 * k, 16)] = r_i[pl.ds(16 * k, 16)]
        return 0

    lax.fori_loop(0, ROWS_PER, scan_row, 0)

    copies = []
    for g in range(ROWS_PER * K // 128):
        sl = pl.ds(g * 128, 128)
        copies.append(pltpu.async_copy(lbl_hbm.at[idxs.at[sl]], lblv.at[sl], sem))
        copies.append(pltpu.async_copy(w_hbm.at[idxs.at[sl]], wvals.at[sl], sem2))
    for cp in copies:
        cp.wait()

    prev_idx = jnp.maximum(iota16 - 1, 0)
    next_idx = jnp.minimum(iota16 + 1, 15)

    def finalize_row(rr, _):
        def zero_body(j, _):
            p_all[pl.ds(rr * CPAD + 16 * j, 16)] = jnp.zeros((16,), jnp.float32)
            return 0

        lax.fori_loop(0, CPAD // 16, zero_body, 0)
        pbase16 = jnp.full((16,), rr * CPAD, jnp.int32)
        for k in range(4):
            off = pl.ds(rr * K + 16 * k, 16)
            e = jnp.exp(wvals[off] - dvals[off] * GC)
            ls, es = plsc.sort_key_val(lblv[off], e)
            cs = plsc.cumsum(es)
            pl_ = jnp.take_along_axis(ls, prev_idx, axis=0,
                                      mode="promise_in_bounds")
            nl_ = jnp.take_along_axis(ls, next_idx, axis=0,
                                      mode="promise_in_bounds")
            is_start = (iota16 == 0) | (ls != pl_)
            is_end = (iota16 == 15) | (ls != nl_)
            pre = jnp.where(iota16 == 0, 0.0,
                            jnp.take_along_axis(cs, prev_idx, axis=0,
                                                mode="promise_in_bounds"))
            plsc.addupdate_scatter(p_all, [pbase16 + ls], cs, mask=is_end)
            plsc.addupdate_scatter(p_all, [pbase16 + ls], -pre, mask=is_start)
        return 0

    lax.fori_loop(0, ROWS_PER, finalize_row, 0)
    pltpu.sync_copy(p_all, out_hbm.at[pl.ds(wid * ROWS_PER * CPAD, ROWS_PER * CPAD)])


_sc_topk = functools.partial(
    pl.kernel,
    out_type=jax.ShapeDtypeStruct((B * CPAD,), jnp.float32),
    mesh=plsc.VectorSubcoreMesh(core_axis_name="c", subcore_axis_name="s"),
    compiler_params=pltpu.CompilerParams(needs_layout_passes=False),
    scratch_types=[
        pltpu.VMEM((NGPAD,), jnp.float32),
        pltpu.VMEM((K, 128), jnp.float32),
        pltpu.VMEM((K,), jnp.int32),
        pltpu.VMEM((K,), jnp.int32),
        pltpu.VMEM((CAP,), jnp.float32),
        pltpu.VMEM((CAP,), jnp.int32),
        pltpu.VMEM((K,), jnp.float32),
        pltpu.VMEM((K,), jnp.int32),
        pltpu.VMEM((ROWS_PER * K,), jnp.float32),
        pltpu.VMEM((ROWS_PER * K,), jnp.int32),
        pltpu.VMEM((ROWS_PER * K,), jnp.int32),
        pltpu.VMEM((ROWS_PER * K,), jnp.float32),
        pltpu.VMEM((ROWS_PER * CPAD,), jnp.float32),
        pltpu.SMEM((1,), jnp.float32),
        pltpu.SMEM((1,), jnp.int32),
        pltpu.SemaphoreType.DMA,
        pltpu.SemaphoreType.DMA,
    ],
)(_sc_body)


def _norm_kernel(p_ref, o_ref):
    p = p_ref[...]
    p = jnp.where(p == 0.0, 1e-10, p)
    p = p / jnp.sum(p, axis=1, keepdims=True)
    o_ref[...] = jnp.log(p)


def kernel(features, centres, centre_labels, weight):
    sq, gmin = pl.pallas_call(
        _dist_kernel,
        grid=(NMT,),
        in_specs=[
            pl.BlockSpec((B, D), lambda m: (0, 0)),
            pl.BlockSpec((MTILE, D), lambda m: (m, 0)),
        ],
        out_specs=[
            pl.BlockSpec((B, MTILE), lambda m: (0, m)),
            pl.BlockSpec((1, B, 8), lambda m: (m, 0, 0)),
        ],
        out_shape=[
            jax.ShapeDtypeStruct((B, MP), jnp.float32),
            jax.ShapeDtypeStruct((NMT, B, 8), jnp.float32),
        ],
    )(features, centres)
    sqtab = sq.reshape(B * NG, 128)
    gmin2 = jnp.concatenate(
        [gmin.transpose(1, 0, 2).reshape(B, NG),
         jnp.full((B, NGPAD - NG), jnp.inf, jnp.float32)], axis=1)
    pflat = _sc_topk(sqtab, gmin2, centre_labels, weight)
    p = pflat.reshape(B, CPAD)[:, :NCLS]
    out = pl.pallas_call(
        _norm_kernel,
        out_shape=jax.ShapeDtypeStruct((B, NCLS), jnp.float32),
    )(p)
    return out


# speedup vs baseline: 137.5341x; 1.5487x over previous
import functools

import jax
import jax.numpy as jnp
from jax import lax
from jax.experimental import pallas as pl
from jax.experimental.pallas import tpu as pltpu
from jax.experimental.pallas import tpu_sc as plsc

B = 1024
D = 128
M = 100000
MTILE = 1024
NMT = 98
MP = MTILE * NMT
NG = MP // 128
NGPAD = 1024
K = 64
NCLS = 1000
CPAD = 1008
GC = 0.5
NWORK = 32
ROWS_PER = B // NWORK
CAP = 192


def _dist_kernel(f_ref, c_ref, o_ref, g_ref):
    f = f_ref[...]
    c = c_ref[...]
    dot = lax.dot_general(f, c, (((1,), (1,)), ((), ())),
                          preferred_element_type=jnp.float32)
    x2 = jnp.sum(f * f, axis=1, keepdims=True)
    c2 = jnp.sum(c * c, axis=1)[None, :]
    sq = jnp.maximum(x2 + c2 - 2.0 * dot, 0.0)
    col = lax.broadcasted_iota(jnp.int32, (B, MTILE), 1) + pl.program_id(0) * MTILE
    sq = jnp.where(col >= M, jnp.inf, sq)
    mins = []
    for g in range(8):
        blk = sq[:, 128 * g:128 * (g + 1)]
        o_ref[g, :, :] = blk
        mins.append(jnp.min(blk, axis=1, keepdims=True))
    g_ref[...] = jnp.concatenate(mins, axis=1)[None]


def _sort16(v, i):
    return plsc.sort_key_val(v, i)


def _minpair(av, ai, bv, bi):
    m = av <= bv
    return jnp.where(m, av, bv), jnp.where(m, ai, bi)


def _maxpair(av, ai, bv, bi):
    m = av <= bv
    return jnp.where(m, bv, av), jnp.where(m, bi, ai)


def _rev(x):
    return lax.rev(x, (0,))


def _merge2(a, b):
    bv, bi = _rev(b[0]), _rev(b[1])
    lv, li = _minpair(a[0], a[1], bv, bi)
    hv, hi = _maxpair(a[0], a[1], bv, bi)
    return _sort16(lv, li), _sort16(hv, hi)


def _sort64(vs, is_):
    s = [_sort16(vs[k], is_[k]) for k in range(4)]
    a0, a1 = _merge2(s[0], s[1])
    a2, a3 = _merge2(s[2], s[3])
    r3 = (_rev(a3[0]), _rev(a3[1]))
    r2 = (_rev(a2[0]), _rev(a2[1]))
    l0 = _minpair(*a0, *r3)
    h0 = _maxpair(*a0, *r3)
    l1 = _minpair(*a1, *r2)
    h1 = _maxpair(*a1, *r2)
    c0 = _minpair(*l0, *l1)
    c1 = _maxpair(*l0, *l1)
    d0 = _minpair(*h0, *h1)
    d1 = _maxpair(*h0, *h1)
    return [_sort16(*c0), _sort16(*c1), _sort16(*d0), _sort16(*d1)]


def _merge_keep64(r, s):
    l = []
    for k in range(4):
        srv, sri = _rev(s[3 - k][0]), _rev(s[3 - k][1])
        l.append(_minpair(r[k][0], r[k][1], srv, sri))
    a0 = _minpair(*l[0], *l[2])
    a2 = _maxpair(*l[0], *l[2])
    a1 = _minpair(*l[1], *l[3])
    a3 = _maxpair(*l[1], *l[3])
    b0 = _minpair(*a0, *a1)
    b1 = _maxpair(*a0, *a1)
    b2 = _minpair(*a2, *a3)
    b3 = _maxpair(*a2, *a3)
    return [_sort16(*b0), _sort16(*b1), _sort16(*b2), _sort16(*b3)]


def _sc_body(sqtab_hbm, gmin_hbm, lbl_hbm, w_hbm, out_hbm,
             gbuf, hot, gids, idxdma, cand_v, cand_i, r_v, r_i,
             dvals, idxs, lblv, wvals, p_all, t_ref, cnt_ref, sem, sem2):
    wid = lax.axis_index("s") * 2 + lax.axis_index("c")
    iota16 = lax.broadcasted_iota(jnp.int32, (16,), 0)
    inf16 = jnp.full((16,), jnp.inf, jnp.float32)
    zero16i = jnp.zeros((16,), jnp.int32)

    def compact():
        cnt16 = jnp.full((16,), cnt_ref[0], jnp.int32)
        rr_ = [(r_v[pl.ds(16 * k, 16)], r_i[pl.ds(16 * k, 16)]) for k in range(4)]
        for blk in range(3):
            sv = []
            si = []
            for k in range(4):
                pos = jnp.full((16,), blk * 64 + 16 * k, jnp.int32) + iota16
                valid = pos < cnt16
                sv.append(jnp.where(valid, cand_v[pl.ds(blk * 64 + 16 * k, 16)],
                                    inf16))
                si.append(cand_i[pl.ds(blk * 64 + 16 * k, 16)])
            rr_ = _merge_keep64(rr_, _sort64(sv, si))
        for k in range(4):
            r_v[pl.ds(16 * k, 16)] = rr_[k][0]
            r_i[pl.ds(16 * k, 16)] = rr_[k][1]
        t_ref[0] = jnp.max(rr_[3][0])
        cnt_ref[0] = 0

    def reset_r():
        for k in range(4):
            r_v[pl.ds(16 * k, 16)] = inf16
            r_i[pl.ds(16 * k, 16)] = zero16i
        cnt_ref[0] = 0

    def group_step(vs, idxvecs):
        m01 = jnp.minimum(vs[0], vs[1])
        m23 = jnp.minimum(vs[2], vs[3])
        m45 = jnp.minimum(vs[4], vs[5])
        m67 = jnp.minimum(vs[6], vs[7])
        mtree = jnp.minimum(jnp.minimum(m01, m23), jnp.minimum(m45, m67))
        mn = jnp.min(mtree)

        @pl.when(mn <= t_ref[0])
        def _():
            @pl.when(cnt_ref[0] > 64)
            def _():
                compact()

            t16 = jnp.full((16,), t_ref[0], jnp.float32)
            for k in range(8):
                mk = vs[k] <= t16
                cnt = cnt_ref[0]
                plsc.store_compressed(cand_v.at[pl.ds(cnt, 16)], vs[k], mask=mk)
                plsc.store_compressed(cand_i.at[pl.ds(cnt, 16)], idxvecs[k],
                                      mask=mk)
                cnt_ref[0] = cnt + jnp.max(plsc.all_reduce_population_count(mk))

    def scan_row(rr, _):
        row = wid * ROWS_PER + rr
        pltpu.sync_copy(gmin_hbm.at[row], gbuf)
        reset_r()
        t_ref[0] = jnp.inf

        def mini_group(g, _):
            base = g * 128
            vs = [gbuf[pl.ds(base + 16 * k, 16)] for k in range(8)]
            ivs = [jnp.full((16,), base + 16 * k, jnp.int32) + iota16
                   for k in range(8)]
            group_step(vs, ivs)
            return 0

        lax.fori_loop(0, NGPAD // 128, mini_group, 0)
        compact()

        row16 = jnp.full((16,), row, jnp.int32)
        for k in range(4):
            gv = r_i[pl.ds(16 * k, 16)]
            gids[pl.ds(16 * k, 16)] = gv
            idxdma[pl.ds(16 * k, 16)] = gv * B + row16
        pltpu.async_copy(sqtab_hbm.at[idxdma], hot, sem).wait()

        reset_r()

        def hot_group(s, _):
            gidv = plsc.load_gather(gids, [jnp.full((16,), s, jnp.int32)])
            base16 = gidv * 128
            vs = [hot[s, pl.ds(16 * k, 16)] for k in range(8)]
            ivs = [base16 + (jnp.full((16,), 16 * k, jnp.int32) + iota16)
                   for k in range(8)]
            group_step(vs, ivs)
            return 0

        lax.fori_loop(0, K, hot_group, 0)
        compact()
        for k in range(4):
            dvals[pl.ds(rr * K + 16 * k, 16)] = r_v[pl.ds(16 * k, 16)]
            idxs[pl.ds(rr * K + 16 * k, 16)] = r_i[pl.ds(16 * k, 16)]
        return 0

    lax.fori_loop(0, ROWS_PER, scan_row, 0)

    copies = []
    for g in range(ROWS_PER * K // 128):
        sl = pl.ds(g * 128, 128)
        copies.append(pltpu.async_copy(lbl_hbm.at[idxs.at[sl]], lblv.at[sl], sem))
        copies.append(pltpu.async_copy(w_hbm.at[idxs.at[sl]], wvals.at[sl], sem2))
    for cp in copies:
        cp.wait()

    prev_idx = jnp.maximum(iota16 - 1, 0)
    next_idx = jnp.minimum(iota16 + 1, 15)

    def finalize_row(rr, _):
        def zero_body(j, _):
            p_all[pl.ds(rr * CPAD + 16 * j, 16)] = jnp.zeros((16,), jnp.float32)
            return 0

        lax.fori_loop(0, CPAD // 16, zero_body, 0)
        pbase16 = jnp.full((16,), rr * CPAD, jnp.int32)
        for k in range(4):
            off = pl.ds(rr * K + 16 * k, 16)
            e = jnp.exp(wvals[off] - dvals[off] * GC)
            ls, es = plsc.sort_key_val(lblv[off], e)
            cs = plsc.cumsum(es)
            pl_ = jnp.take_along_axis(ls, prev_idx, axis=0,
                                      mode="promise_in_bounds")
            nl_ = jnp.take_along_axis(ls, next_idx, axis=0,
                                      mode="promise_in_bounds")
            is_start = (iota16 == 0) | (ls != pl_)
            is_end = (iota16 == 15) | (ls != nl_)
            pre = jnp.where(iota16 == 0, 0.0,
                            jnp.take_along_axis(cs, prev_idx, axis=0,
                                                mode="promise_in_bounds"))
            plsc.addupdate_scatter(p_all, [pbase16 + ls], cs, mask=is_end)
            plsc.addupdate_scatter(p_all, [pbase16 + ls], -pre, mask=is_start)
        return 0

    lax.fori_loop(0, ROWS_PER, finalize_row, 0)
    pltpu.sync_copy(p_all, out_hbm.at[pl.ds(wid * ROWS_PER * CPAD, ROWS_PER * CPAD)])


_sc_topk = functools.partial(
    pl.kernel,
    out_type=jax.ShapeDtypeStruct((B * CPAD,), jnp.float32),
    mesh=plsc.VectorSubcoreMesh(core_axis_name="c", subcore_axis_name="s"),
    compiler_params=pltpu.CompilerParams(needs_layout_passes=False),
    scratch_types=[
        pltpu.VMEM((NGPAD,), jnp.float32),
        pltpu.VMEM((K, 128), jnp.float32),
        pltpu.VMEM((K,), jnp.int32),
        pltpu.VMEM((K,), jnp.int32),
        pltpu.VMEM((CAP,), jnp.float32),
        pltpu.VMEM((CAP,), jnp.int32),
        pltpu.VMEM((K,), jnp.float32),
        pltpu.VMEM((K,), jnp.int32),
        pltpu.VMEM((ROWS_PER * K,), jnp.float32),
        pltpu.VMEM((ROWS_PER * K,), jnp.int32),
        pltpu.VMEM((ROWS_PER * K,), jnp.int32),
        pltpu.VMEM((ROWS_PER * K,), jnp.float32),
        pltpu.VMEM((ROWS_PER * CPAD,), jnp.float32),
        pltpu.SMEM((1,), jnp.float32),
        pltpu.SMEM((1,), jnp.int32),
        pltpu.SemaphoreType.DMA,
        pltpu.SemaphoreType.DMA,
    ],
)(_sc_body)


def _norm_kernel(p_ref, o_ref):
    p = p_ref[...][:, :NCLS]
    p = jnp.where(p == 0.0, 1e-10, p)
    p = p / jnp.sum(p, axis=1, keepdims=True)
    o_ref[...] = jnp.log(p)


def kernel(features, centres, centre_labels, weight):
    sq, gmin = pl.pallas_call(
        _dist_kernel,
        grid=(NMT,),
        in_specs=[
            pl.BlockSpec((B, D), lambda m: (0, 0)),
            pl.BlockSpec((MTILE, D), lambda m: (m, 0)),
        ],
        out_specs=[
            pl.BlockSpec((8, B, 128), lambda m: (m, 0, 0)),
            pl.BlockSpec((1, B, 8), lambda m: (m, 0, 0)),
        ],
        out_shape=[
            jax.ShapeDtypeStruct((NG, B, 128), jnp.float32),
            jax.ShapeDtypeStruct((NMT, B, 8), jnp.float32),
        ],
    )(features, centres)
    sqtab = sq.reshape(NG * B, 128)
    gmin2 = jnp.concatenate(
        [gmin.transpose(1, 0, 2).reshape(B, NG),
         jnp.full((B, NGPAD - NG), jnp.inf, jnp.float32)], axis=1)
    pflat = _sc_topk(sqtab, gmin2, centre_labels, weight)
    out = pl.pallas_call(
        _norm_kernel,
        out_shape=jax.ShapeDtypeStruct((B, NCLS), jnp.float32),
    )(pflat.reshape(B, CPAD))
    return out


# speedup vs baseline: 145.8917x; 1.0608x over previous
import functools

import jax
import jax.numpy as jnp
from jax import lax
from jax.experimental import pallas as pl
from jax.experimental.pallas import tpu as pltpu
from jax.experimental.pallas import tpu_sc as plsc

B = 1024
D = 128
M = 100000
MTILE = 1024
NMT = 98
MP = MTILE * NMT
NG = MP // 128
NGPAD = 1024
K = 64
NCLS = 1000
CPAD = 1008
GC = 0.5
NWORK = 32
ROWS_PER = B // NWORK
CAP = 192


def _dist_kernel(f_ref, c_ref, o_ref, g_ref):
    f = f_ref[...]
    c = c_ref[...]
    dot = lax.dot_general(f, c, (((1,), (1,)), ((), ())),
                          preferred_element_type=jnp.float32)
    x2 = jnp.sum(f * f, axis=1, keepdims=True)
    c2 = jnp.sum(c * c, axis=1)[None, :]
    sq = jnp.maximum(x2 + c2 - 2.0 * dot, 0.0)
    col = lax.broadcasted_iota(jnp.int32, (B, MTILE), 1) + pl.program_id(0) * MTILE
    sq = jnp.where(col >= M, jnp.inf, sq)
    mins = []
    for g in range(8):
        blk = sq[:, 128 * g:128 * (g + 1)]
        o_ref[g, :, :] = blk
        mins.append(jnp.min(blk, axis=1, keepdims=True))
    g_ref[...] = jnp.concatenate(mins, axis=1)[None]


def _sort16(v, i):
    return plsc.sort_key_val(v, i)


def _minpair(av, ai, bv, bi):
    m = av <= bv
    return jnp.where(m, av, bv), jnp.where(m, ai, bi)


def _maxpair(av, ai, bv, bi):
    m = av <= bv
    return jnp.where(m, bv, av), jnp.where(m, bi, ai)


def _rev(x):
    return lax.rev(x, (0,))


def _merge2(a, b):
    bv, bi = _rev(b[0]), _rev(b[1])
    lv, li = _minpair(a[0], a[1], bv, bi)
    hv, hi = _maxpair(a[0], a[1], bv, bi)
    return _sort16(lv, li), _sort16(hv, hi)


def _sort64(vs, is_):
    s = [_sort16(vs[k], is_[k]) for k in range(4)]
    a0, a1 = _merge2(s[0], s[1])
    a2, a3 = _merge2(s[2], s[3])
    r3 = (_rev(a3[0]), _rev(a3[1]))
    r2 = (_rev(a2[0]), _rev(a2[1]))
    l0 = _minpair(*a0, *r3)
    h0 = _maxpair(*a0, *r3)
    l1 = _minpair(*a1, *r2)
    h1 = _maxpair(*a1, *r2)
    c0 = _minpair(*l0, *l1)
    c1 = _maxpair(*l0, *l1)
    d0 = _minpair(*h0, *h1)
    d1 = _maxpair(*h0, *h1)
    return [_sort16(*c0), _sort16(*c1), _sort16(*d0), _sort16(*d1)]


def _merge_keep64(r, s):
    l = []
    for k in range(4):
        srv, sri = _rev(s[3 - k][0]), _rev(s[3 - k][1])
        l.append(_minpair(r[k][0], r[k][1], srv, sri))
    a0 = _minpair(*l[0], *l[2])
    a2 = _maxpair(*l[0], *l[2])
    a1 = _minpair(*l[1], *l[3])
    a3 = _maxpair(*l[1], *l[3])
    b0 = _minpair(*a0, *a1)
    b1 = _maxpair(*a0, *a1)
    b2 = _minpair(*a2, *a3)
    b3 = _maxpair(*a2, *a3)
    return [_sort16(*b0), _sort16(*b1), _sort16(*b2), _sort16(*b3)]


def _sc_body(sqtab_hbm, gmin_hbm, lbl_hbm, w_hbm, out_hbm,
             gbuf, hot0, hot1, gids0, gids1, idx0, idx1,
             cand_v, cand_i, r_v, r_i,
             dvals, idxs, lblv, wvals, p_all, t_ref, t0s, cnt_ref,
             gsem0, gsem1, sem, sem2):
    wid = lax.axis_index("s") * 2 + lax.axis_index("c")
    iota16 = lax.broadcasted_iota(jnp.int32, (16,), 0)
    inf16 = jnp.full((16,), jnp.inf, jnp.float32)
    zero16i = jnp.zeros((16,), jnp.int32)

    def compact():
        cnt = cnt_ref[0]
        cnt16 = jnp.full((16,), cnt, jnp.int32)

        def do_block(blk):
            rr_ = [(r_v[pl.ds(16 * k, 16)], r_i[pl.ds(16 * k, 16)])
                   for k in range(4)]
            sv = []
            si = []
            for k in range(4):
                pos = jnp.full((16,), blk * 64 + 16 * k, jnp.int32) + iota16
                valid = pos < cnt16
                sv.append(jnp.where(valid, cand_v[pl.ds(blk * 64 + 16 * k, 16)],
                                    inf16))
                si.append(cand_i[pl.ds(blk * 64 + 16 * k, 16)])
            rr_ = _merge_keep64(rr_, _sort64(sv, si))
            for k in range(4):
                r_v[pl.ds(16 * k, 16)] = rr_[k][0]
                r_i[pl.ds(16 * k, 16)] = rr_[k][1]

        do_block(0)

        @pl.when(cnt > 64)
        def _():
            do_block(1)

        @pl.when(cnt > 128)
        def _():
            do_block(2)

        t_ref[0] = jnp.max(r_v[pl.ds(48, 16)])
        cnt_ref[0] = 0

    def reset_r():
        for k in range(4):
            r_v[pl.ds(16 * k, 16)] = inf16
            r_i[pl.ds(16 * k, 16)] = zero16i
        cnt_ref[0] = 0

    def group_step(vs, idxvecs):
        m01 = jnp.minimum(vs[0], vs[1])
        m23 = jnp.minimum(vs[2], vs[3])
        m45 = jnp.minimum(vs[4], vs[5])
        m67 = jnp.minimum(vs[6], vs[7])
        mtree = jnp.minimum(jnp.minimum(m01, m23), jnp.minimum(m45, m67))
        mn = jnp.min(mtree)

        @pl.when(mn <= t_ref[0])
        def _():
            @pl.when(cnt_ref[0] > 64)
            def _():
                compact()

            t16 = jnp.full((16,), t_ref[0], jnp.float32)
            for k in range(8):
                mk = vs[k] <= t16
                cnt = cnt_ref[0]
                plsc.store_compressed(cand_v.at[pl.ds(cnt, 16)], vs[k], mask=mk)
                plsc.store_compressed(cand_i.at[pl.ds(cnt, 16)], idxvecs[k],
                                      mask=mk)
                cnt_ref[0] = cnt + jnp.max(plsc.all_reduce_population_count(mk))

    def mini_and_fire(row, hotr, gidsr, idxr, gsem, slot):
        pltpu.sync_copy(gmin_hbm.at[row], gbuf)
        reset_r()
        t_ref[0] = jnp.inf

        def mini_group(g, _):
            base = g * 128
            vs = [gbuf[pl.ds(base + 16 * k, 16)] for k in range(8)]
            ivs = [jnp.full((16,), base + 16 * k, jnp.int32) + iota16
                   for k in range(8)]
            group_step(vs, ivs)
            return 0

        lax.fori_loop(0, NGPAD // 128, mini_group, 0)
        compact()
        row16 = jnp.full((16,), row, jnp.int32)
        for k in range(4):
            gv = r_i[pl.ds(16 * k, 16)]
            gidsr[pl.ds(16 * k, 16)] = gv
            idxr[pl.ds(16 * k, 16)] = gv * B + row16
        t0s[slot] = t_ref[0]
        pltpu.async_copy(sqtab_hbm.at[idxr], hotr, gsem)

    def hot_scan(rr, hotr, gidsr, idxr, gsem, slot):
        pltpu.make_async_copy(sqtab_hbm.at[idxr], hotr, gsem).wait()
        t_ref[0] = t0s[slot]
        reset_r()

        def hot_group(s, _):
            gidv = plsc.load_gather(gidsr, [jnp.full((16,), s, jnp.int32)])
            base16 = gidv * 128
            vs = [hotr[s, pl.ds(16 * k, 16)] for k in range(8)]
            ivs = [base16 + (jnp.full((16,), 16 * k, jnp.int32) + iota16)
                   for k in range(8)]
            group_step(vs, ivs)
            return 0

        lax.fori_loop(0, K, hot_group, 0)
        compact()
        for k in range(4):
            dvals[pl.ds(rr * K + 16 * k, 16)] = r_v[pl.ds(16 * k, 16)]
            idxs[pl.ds(rr * K + 16 * k, 16)] = r_i[pl.ds(16 * k, 16)]

    mini_and_fire(wid * ROWS_PER, hot0, gids0, idx0, gsem0, 0)

    def scan_iter(rr, _):
        row = wid * ROWS_PER + rr
        par = lax.rem(rr, 2)

        @pl.when(par == 0)
        def _():
            @pl.when(rr + 1 < ROWS_PER)
            def _():
                mini_and_fire(row + 1, hot1, gids1, idx1, gsem1, 1)

            hot_scan(rr, hot0, gids0, idx0, gsem0, 0)

        @pl.when(par == 1)
        def _():
            @pl.when(rr + 1 < ROWS_PER)
            def _():
                mini_and_fire(row + 1, hot0, gids0, idx0, gsem0, 0)

            hot_scan(rr, hot1, gids1, idx1, gsem1, 1)

        return 0

    lax.fori_loop(0, ROWS_PER, scan_iter, 0)

    copies = []
    for g in range(ROWS_PER * K // 128):
        sl = pl.ds(g * 128, 128)
        copies.append(pltpu.async_copy(lbl_hbm.at[idxs.at[sl]], lblv.at[sl], sem))
        copies.append(pltpu.async_copy(w_hbm.at[idxs.at[sl]], wvals.at[sl], sem2))
    for cp in copies:
        cp.wait()

    prev_idx = jnp.maximum(iota16 - 1, 0)
    next_idx = jnp.minimum(iota16 + 1, 15)

    def finalize_row(rr, _):
        def zero_body(j, _):
            p_all[pl.ds(rr * CPAD + 16 * j, 16)] = jnp.zeros((16,), jnp.float32)
            return 0

        lax.fori_loop(0, CPAD // 16, zero_body, 0)
        pbase16 = jnp.full((16,), rr * CPAD, jnp.int32)
        for k in range(4):
            off = pl.ds(rr * K + 16 * k, 16)
            e = jnp.exp(wvals[off] - dvals[off] * GC)
            ls, es = plsc.sort_key_val(lblv[off], e)
            cs = plsc.cumsum(es)
            pl_ = jnp.take_along_axis(ls, prev_idx, axis=0,
                                      mode="promise_in_bounds")
            nl_ = jnp.take_along_axis(ls, next_idx, axis=0,
                                      mode="promise_in_bounds")
            is_start = (iota16 == 0) | (ls != pl_)
            is_end = (iota16 == 15) | (ls != nl_)
            pre = jnp.where(iota16 == 0, 0.0,
                            jnp.take_along_axis(cs, prev_idx, axis=0,
                                                mode="promise_in_bounds"))
            plsc.addupdate_scatter(p_all, [pbase16 + ls], cs, mask=is_end)
            plsc.addupdate_scatter(p_all, [pbase16 + ls], -pre, mask=is_start)
        return 0

    lax.fori_loop(0, ROWS_PER, finalize_row, 0)
    pltpu.sync_copy(p_all, out_hbm.at[pl.ds(wid * ROWS_PER * CPAD, ROWS_PER * CPAD)])


_sc_topk = functools.partial(
    pl.kernel,
    out_type=jax.ShapeDtypeStruct((B * CPAD,), jnp.float32),
    mesh=plsc.VectorSubcoreMesh(core_axis_name="c", subcore_axis_name="s"),
    compiler_params=pltpu.CompilerParams(needs_layout_passes=False),
    scratch_types=[
        pltpu.VMEM((NGPAD,), jnp.float32),
        pltpu.VMEM((K, 128), jnp.float32),
        pltpu.VMEM((K, 128), jnp.float32),
        pltpu.VMEM((K,), jnp.int32),
        pltpu.VMEM((K,), jnp.int32),
        pltpu.VMEM((K,), jnp.int32),
        pltpu.VMEM((K,), jnp.int32),
        pltpu.VMEM((CAP,), jnp.float32),
        pltpu.VMEM((CAP,), jnp.int32),
        pltpu.VMEM((K,), jnp.float32),
        pltpu.VMEM((K,), jnp.int32),
        pltpu.VMEM((ROWS_PER * K,), jnp.float32),
        pltpu.VMEM((ROWS_PER * K,), jnp.int32),
        pltpu.VMEM((ROWS_PER * K,), jnp.int32),
        pltpu.VMEM((ROWS_PER * K,), jnp.float32),
        pltpu.VMEM((ROWS_PER * CPAD,), jnp.float32),
        pltpu.SMEM((1,), jnp.float32),
        pltpu.SMEM((2,), jnp.float32),
        pltpu.SMEM((1,), jnp.int32),
        pltpu.SemaphoreType.DMA,
        pltpu.SemaphoreType.DMA,
        pltpu.SemaphoreType.DMA,
        pltpu.SemaphoreType.DMA,
    ],
)(_sc_body)


def _norm_kernel(p_ref, o_ref):
    p = p_ref[...][:, :NCLS]
    p = jnp.where(p == 0.0, 1e-10, p)
    p = p / jnp.sum(p, axis=1, keepdims=True)
    o_ref[...] = jnp.log(p)


def kernel(features, centres, centre_labels, weight):
    sq, gmin = pl.pallas_call(
        _dist_kernel,
        grid=(NMT,),
        in_specs=[
            pl.BlockSpec((B, D), lambda m: (0, 0)),
            pl.BlockSpec((MTILE, D), lambda m: (m, 0)),
        ],
        out_specs=[
            pl.BlockSpec((8, B, 128), lambda m: (m, 0, 0)),
            pl.BlockSpec((1, B, 8), lambda m: (m, 0, 0)),
        ],
        out_shape=[
            jax.ShapeDtypeStruct((NG, B, 128), jnp.float32),
            jax.ShapeDtypeStruct((NMT, B, 8), jnp.float32),
        ],
    )(features, centres)
    sqtab = sq.reshape(NG * B, 128)
    gmin2 = jnp.concatenate(
        [gmin.transpose(1, 0, 2).reshape(B, NG),
         jnp.full((B, NGPAD - NG), jnp.inf, jnp.float32)], axis=1)
    pflat = _sc_topk(sqtab, gmin2, centre_labels, weight)
    out = pl.pallas_call(
        _norm_kernel,
        out_shape=jax.ShapeDtypeStruct((B, NCLS), jnp.float32),
    )(pflat.reshape(B, CPAD))
    return out


# speedup vs baseline: 163.5447x; 1.1210x over previous
import functools

import jax
import jax.numpy as jnp
from jax import lax
from jax.experimental import pallas as pl
from jax.experimental.pallas import tpu as pltpu
from jax.experimental.pallas import tpu_sc as plsc

B = 1024
D = 128
M = 100000
MTILE = 2048
NMT = 49
MP = MTILE * NMT
NG = MP // 128
NGPAD = 1024
K = 64
NCLS = 1000
CPAD = 1008
GC = 0.5
NWORK = 32
ROWS_PER = B // NWORK
CAP = 192


def _dist_kernel(f_ref, c_ref, o_ref, g_ref):
    f = f_ref[...]
    c = c_ref[...]
    dot = lax.dot_general(f, c, (((1,), (1,)), ((), ())),
                          preferred_element_type=jnp.float32)
    x2 = jnp.sum(f * f, axis=1, keepdims=True)
    c2 = jnp.sum(c * c, axis=1)[None, :]
    sq = jnp.maximum(x2 + c2 - 2.0 * dot, 0.0)
    col = lax.broadcasted_iota(jnp.int32, (B, MTILE), 1) + pl.program_id(0) * MTILE
    sq = jnp.where(col >= M, jnp.inf, sq)
    mins = []
    for g in range(MTILE // 128):
        blk = sq[:, 128 * g:128 * (g + 1)]
        o_ref[g, :, :] = blk
        mins.append(jnp.min(blk, axis=1, keepdims=True))
    g_ref[...] = jnp.concatenate(mins, axis=1)[None]


def _sort16(v, i):
    return plsc.sort_key_val(v, i)


def _minpair(av, ai, bv, bi):
    m = av <= bv
    return jnp.where(m, av, bv), jnp.where(m, ai, bi)


def _maxpair(av, ai, bv, bi):
    m = av <= bv
    return jnp.where(m, bv, av), jnp.where(m, bi, ai)


def _rev(x):
    return lax.rev(x, (0,))


def _merge2(a, b):
    bv, bi = _rev(b[0]), _rev(b[1])
    lv, li = _minpair(a[0], a[1], bv, bi)
    hv, hi = _maxpair(a[0], a[1], bv, bi)
    return _sort16(lv, li), _sort16(hv, hi)


def _sort64(vs, is_):
    s = [_sort16(vs[k], is_[k]) for k in range(4)]
    a0, a1 = _merge2(s[0], s[1])
    a2, a3 = _merge2(s[2], s[3])
    r3 = (_rev(a3[0]), _rev(a3[1]))
    r2 = (_rev(a2[0]), _rev(a2[1]))
    l0 = _minpair(*a0, *r3)
    h0 = _maxpair(*a0, *r3)
    l1 = _minpair(*a1, *r2)
    h1 = _maxpair(*a1, *r2)
    c0 = _minpair(*l0, *l1)
    c1 = _maxpair(*l0, *l1)
    d0 = _minpair(*h0, *h1)
    d1 = _maxpair(*h0, *h1)
    return [_sort16(*c0), _sort16(*c1), _sort16(*d0), _sort16(*d1)]


def _merge_keep64(r, s):
    l = []
    for k in range(4):
        srv, sri = _rev(s[3 - k][0]), _rev(s[3 - k][1])
        l.append(_minpair(r[k][0], r[k][1], srv, sri))
    a0 = _minpair(*l[0], *l[2])
    a2 = _maxpair(*l[0], *l[2])
    a1 = _minpair(*l[1], *l[3])
    a3 = _maxpair(*l[1], *l[3])
    b0 = _minpair(*a0, *a1)
    b1 = _maxpair(*a0, *a1)
    b2 = _minpair(*a2, *a3)
    b3 = _maxpair(*a2, *a3)
    return [_sort16(*b0), _sort16(*b1), _sort16(*b2), _sort16(*b3)]


def _sc_body(sqtab_hbm, gmin_hbm, lbl_hbm, w_hbm, out_hbm,
             gbuf, hot0, hot1, gids0, gids1, idx0, idx1,
             cand_v, cand_i, r_v, r_i,
             dvals, idxs, lblv, wvals, p_all, t_ref, t0s, cnt_ref,
             gsem0, gsem1, sem, sem2):
    wid = lax.axis_index("s") * 2 + lax.axis_index("c")
    iota16 = lax.broadcasted_iota(jnp.int32, (16,), 0)
    inf16 = jnp.full((16,), jnp.inf, jnp.float32)
    zero16i = jnp.zeros((16,), jnp.int32)

    def compact():
        cnt = cnt_ref[0]
        cnt16 = jnp.full((16,), cnt, jnp.int32)

        def do_block(blk):
            rr_ = [(r_v[pl.ds(16 * k, 16)], r_i[pl.ds(16 * k, 16)])
                   for k in range(4)]
            sv = []
            si = []
            for k in range(4):
                pos = jnp.full((16,), blk * 64 + 16 * k, jnp.int32) + iota16
                valid = pos < cnt16
                sv.append(jnp.where(valid, cand_v[pl.ds(blk * 64 + 16 * k, 16)],
                                    inf16))
                si.append(cand_i[pl.ds(blk * 64 + 16 * k, 16)])
            rr_ = _merge_keep64(rr_, _sort64(sv, si))
            for k in range(4):
                r_v[pl.ds(16 * k, 16)] = rr_[k][0]
                r_i[pl.ds(16 * k, 16)] = rr_[k][1]

        do_block(0)

        @pl.when(cnt > 64)
        def _():
            do_block(1)

        @pl.when(cnt > 128)
        def _():
            do_block(2)

        t_ref[0] = jnp.max(r_v[pl.ds(48, 16)])
        cnt_ref[0] = 0

    def reset_r():
        for k in range(4):
            r_v[pl.ds(16 * k, 16)] = inf16
            r_i[pl.ds(16 * k, 16)] = zero16i
        cnt_ref[0] = 0

    def group_step(vs, idxvecs):
        m01 = jnp.minimum(vs[0], vs[1])
        m23 = jnp.minimum(vs[2], vs[3])
        m45 = jnp.minimum(vs[4], vs[5])
        m67 = jnp.minimum(vs[6], vs[7])
        mtree = jnp.minimum(jnp.minimum(m01, m23), jnp.minimum(m45, m67))
        mn = jnp.min(mtree)

        @pl.when(mn <= t_ref[0])
        def _():
            @pl.when(cnt_ref[0] > 64)
            def _():
                compact()

            t16 = jnp.full((16,), t_ref[0], jnp.float32)
            for k in range(8):
                mk = vs[k] <= t16
                cnt = cnt_ref[0]
                plsc.store_compressed(cand_v.at[pl.ds(cnt, 16)], vs[k], mask=mk)
                plsc.store_compressed(cand_i.at[pl.ds(cnt, 16)], idxvecs[k],
                                      mask=mk)
                cnt_ref[0] = cnt + jnp.max(plsc.all_reduce_population_count(mk))

    def mini_and_fire(row, hotr, gidsr, idxr, gsem, slot):
        pltpu.sync_copy(gmin_hbm.at[row], gbuf)
        reset_r()
        t_ref[0] = jnp.inf

        def mini_group(g, _):
            base = g * 128
            vs = [gbuf[pl.ds(base + 16 * k, 16)] for k in range(8)]
            ivs = [jnp.full((16,), base + 16 * k, jnp.int32) + iota16
                   for k in range(8)]
            group_step(vs, ivs)
            return 0

        lax.fori_loop(0, NGPAD // 128, mini_group, 0)
        compact()
        row16 = jnp.full((16,), row, jnp.int32)
        for k in range(4):
            gv = r_i[pl.ds(16 * k, 16)]
            gidsr[pl.ds(16 * k, 16)] = gv
            idxr[pl.ds(16 * k, 16)] = gv * B + row16
        t0s[slot] = t_ref[0]
        pltpu.async_copy(sqtab_hbm.at[idxr], hotr, gsem)

    def hot_scan(rr, hotr, gidsr, idxr, gsem, slot):
        pltpu.make_async_copy(sqtab_hbm.at[idxr], hotr, gsem).wait()
        t_ref[0] = t0s[slot]
        reset_r()

        def hot_group(s, _):
            gidv = plsc.load_gather(gidsr, [jnp.full((16,), s, jnp.int32)])
            base16 = gidv * 128
            vs = [hotr[s, pl.ds(16 * k, 16)] for k in range(8)]
            ivs = [base16 + (jnp.full((16,), 16 * k, jnp.int32) + iota16)
                   for k in range(8)]
            group_step(vs, ivs)
            return 0

        lax.fori_loop(0, K, hot_group, 0)
        compact()
        for k in range(4):
            dvals[pl.ds(rr * K + 16 * k, 16)] = r_v[pl.ds(16 * k, 16)]
            idxs[pl.ds(rr * K + 16 * k, 16)] = r_i[pl.ds(16 * k, 16)]

    mini_and_fire(wid * ROWS_PER, hot0, gids0, idx0, gsem0, 0)

    def scan_iter(rr, _):
        row = wid * ROWS_PER + rr
        par = lax.rem(rr, 2)

        @pl.when(par == 0)
        def _():
            @pl.when(rr + 1 < ROWS_PER)
            def _():
                mini_and_fire(row + 1, hot1, gids1, idx1, gsem1, 1)

            hot_scan(rr, hot0, gids0, idx0, gsem0, 0)

        @pl.when(par == 1)
        def _():
            @pl.when(rr + 1 < ROWS_PER)
            def _():
                mini_and_fire(row + 1, hot0, gids0, idx0, gsem0, 0)

            hot_scan(rr, hot1, gids1, idx1, gsem1, 1)

        return 0

    lax.fori_loop(0, ROWS_PER, scan_iter, 0)

    copies = []
    for g in range(ROWS_PER * K // 128):
        sl = pl.ds(g * 128, 128)
        copies.append(pltpu.async_copy(lbl_hbm.at[idxs.at[sl]], lblv.at[sl], sem))
        copies.append(pltpu.async_copy(w_hbm.at[idxs.at[sl]], wvals.at[sl], sem2))
    for cp in copies:
        cp.wait()

    prev_idx = jnp.maximum(iota16 - 1, 0)
    next_idx = jnp.minimum(iota16 + 1, 15)

    def finalize_row(rr, _):
        def zero_body(j, _):
            p_all[pl.ds(rr * CPAD + 16 * j, 16)] = jnp.zeros((16,), jnp.float32)
            return 0

        lax.fori_loop(0, CPAD // 16, zero_body, 0)
        pbase16 = jnp.full((16,), rr * CPAD, jnp.int32)
        for k in range(4):
            off = pl.ds(rr * K + 16 * k, 16)
            e = jnp.exp(wvals[off] - dvals[off] * GC)
            ls, es = plsc.sort_key_val(lblv[off], e)
            cs = plsc.cumsum(es)
            pl_ = jnp.take_along_axis(ls, prev_idx, axis=0,
                                      mode="promise_in_bounds")
            nl_ = jnp.take_along_axis(ls, next_idx, axis=0,
                                      mode="promise_in_bounds")
            is_start = (iota16 == 0) | (ls != pl_)
            is_end = (iota16 == 15) | (ls != nl_)
            pre = jnp.where(iota16 == 0, 0.0,
                            jnp.take_along_axis(cs, prev_idx, axis=0,
                                                mode="promise_in_bounds"))
            plsc.addupdate_scatter(p_all, [pbase16 + ls], cs, mask=is_end)
            plsc.addupdate_scatter(p_all, [pbase16 + ls], -pre, mask=is_start)
        return 0

    lax.fori_loop(0, ROWS_PER, finalize_row, 0)
    pltpu.sync_copy(p_all, out_hbm.at[pl.ds(wid * ROWS_PER * CPAD, ROWS_PER * CPAD)])


_sc_topk = functools.partial(
    pl.kernel,
    out_type=jax.ShapeDtypeStruct((B * CPAD,), jnp.float32),
    mesh=plsc.VectorSubcoreMesh(core_axis_name="c", subcore_axis_name="s"),
    compiler_params=pltpu.CompilerParams(needs_layout_passes=False),
    scratch_types=[
        pltpu.VMEM((NGPAD,), jnp.float32),
        pltpu.VMEM((K, 128), jnp.float32),
        pltpu.VMEM((K, 128), jnp.float32),
        pltpu.VMEM((K,), jnp.int32),
        pltpu.VMEM((K,), jnp.int32),
        pltpu.VMEM((K,), jnp.int32),
        pltpu.VMEM((K,), jnp.int32),
        pltpu.VMEM((CAP,), jnp.float32),
        pltpu.VMEM((CAP,), jnp.int32),
        pltpu.VMEM((K,), jnp.float32),
        pltpu.VMEM((K,), jnp.int32),
        pltpu.VMEM((ROWS_PER * K,), jnp.float32),
        pltpu.VMEM((ROWS_PER * K,), jnp.int32),
        pltpu.VMEM((ROWS_PER * K,), jnp.int32),
        pltpu.VMEM((ROWS_PER * K,), jnp.float32),
        pltpu.VMEM((ROWS_PER * CPAD,), jnp.float32),
        pltpu.SMEM((1,), jnp.float32),
        pltpu.SMEM((2,), jnp.float32),
        pltpu.SMEM((1,), jnp.int32),
        pltpu.SemaphoreType.DMA,
        pltpu.SemaphoreType.DMA,
        pltpu.SemaphoreType.DMA,
        pltpu.SemaphoreType.DMA,
    ],
)(_sc_body)


def _norm_kernel(p_ref, o_ref):
    p = p_ref[...][:, :NCLS]
    p = jnp.where(p == 0.0, 1e-10, p)
    p = p / jnp.sum(p, axis=1, keepdims=True)
    o_ref[...] = jnp.log(p)


def kernel(features, centres, centre_labels, weight):
    sq, gmin = pl.pallas_call(
        _dist_kernel,
        grid=(NMT,),
        in_specs=[
            pl.BlockSpec((B, D), lambda m: (0, 0)),
            pl.BlockSpec((MTILE, D), lambda m: (m, 0)),
        ],
        out_specs=[
            pl.BlockSpec((MTILE // 128, B, 128), lambda m: (m, 0, 0)),
            pl.BlockSpec((1, B, MTILE // 128), lambda m: (m, 0, 0)),
        ],
        out_shape=[
            jax.ShapeDtypeStruct((NG, B, 128), jnp.float32),
            jax.ShapeDtypeStruct((NMT, B, MTILE // 128), jnp.float32),
        ],
    )(features, centres)
    sqtab = sq.reshape(NG * B, 128)
    gmin2 = jnp.concatenate(
        [gmin.transpose(1, 0, 2).reshape(B, NG),
         jnp.full((B, NGPAD - NG), jnp.inf, jnp.float32)], axis=1)
    pflat = _sc_topk(sqtab, gmin2, centre_labels, weight)
    out = pl.pallas_call(
        _norm_kernel,
        out_shape=jax.ShapeDtypeStruct((B, NCLS), jnp.float32),
    )(pflat.reshape(B, CPAD))
    return out


# speedup vs baseline: 166.2946x; 1.0168x over previous
import functools

import jax
import jax.numpy as jnp
from jax import lax
from jax.experimental import pallas as pl
from jax.experimental.pallas import tpu as pltpu
from jax.experimental.pallas import tpu_sc as plsc

B = 1024
D = 128
M = 100000
MTILE = 3584
NMT = 28
MP = MTILE * NMT
NG = MP // 128
NGPAD = 1024
K = 64
NCLS = 1000
CPAD = 1008
GC = 0.5
NWORK = 32
ROWS_PER = B // NWORK
CAP = 192


def _dist_kernel(f_ref, c_ref, o_ref, g_ref):
    f = f_ref[...]
    c = c_ref[...]
    dot = lax.dot_general(f, c, (((1,), (1,)), ((), ())),
                          preferred_element_type=jnp.float32)
    x2 = jnp.sum(f * f, axis=1, keepdims=True)
    c2 = jnp.sum(c * c, axis=1)[None, :]
    sq = jnp.maximum(x2 + c2 - 2.0 * dot, 0.0)
    col = lax.broadcasted_iota(jnp.int32, (B, MTILE), 1) + pl.program_id(0) * MTILE
    sq = jnp.where(col >= M, jnp.inf, sq)
    mins = []
    for g in range(MTILE // 128):
        blk = sq[:, 128 * g:128 * (g + 1)]
        o_ref[g, :, :] = blk
        mins.append(jnp.min(blk, axis=1, keepdims=True))
    g_ref[...] = jnp.concatenate(mins, axis=1)[None]


def _sort16(v, i):
    return plsc.sort_key_val(v, i)


def _minpair(av, ai, bv, bi):
    m = av <= bv
    return jnp.where(m, av, bv), jnp.where(m, ai, bi)


def _maxpair(av, ai, bv, bi):
    m = av <= bv
    return jnp.where(m, bv, av), jnp.where(m, bi, ai)


def _rev(x):
    return lax.rev(x, (0,))


def _merge2(a, b):
    bv, bi = _rev(b[0]), _rev(b[1])
    lv, li = _minpair(a[0], a[1], bv, bi)
    hv, hi = _maxpair(a[0], a[1], bv, bi)
    return _sort16(lv, li), _sort16(hv, hi)


def _sort64(vs, is_):
    s = [_sort16(vs[k], is_[k]) for k in range(4)]
    a0, a1 = _merge2(s[0], s[1])
    a2, a3 = _merge2(s[2], s[3])
    r3 = (_rev(a3[0]), _rev(a3[1]))
    r2 = (_rev(a2[0]), _rev(a2[1]))
    l0 = _minpair(*a0, *r3)
    h0 = _maxpair(*a0, *r3)
    l1 = _minpair(*a1, *r2)
    h1 = _maxpair(*a1, *r2)
    c0 = _minpair(*l0, *l1)
    c1 = _maxpair(*l0, *l1)
    d0 = _minpair(*h0, *h1)
    d1 = _maxpair(*h0, *h1)
    return [_sort16(*c0), _sort16(*c1), _sort16(*d0), _sort16(*d1)]


def _merge_keep64(r, s):
    l = []
    for k in range(4):
        srv, sri = _rev(s[3 - k][0]), _rev(s[3 - k][1])
        l.append(_minpair(r[k][0], r[k][1], srv, sri))
    a0 = _minpair(*l[0], *l[2])
    a2 = _maxpair(*l[0], *l[2])
    a1 = _minpair(*l[1], *l[3])
    a3 = _maxpair(*l[1], *l[3])
    b0 = _minpair(*a0, *a1)
    b1 = _maxpair(*a0, *a1)
    b2 = _minpair(*a2, *a3)
    b3 = _maxpair(*a2, *a3)
    return [_sort16(*b0), _sort16(*b1), _sort16(*b2), _sort16(*b3)]


def _sc_body(sqtab_hbm, gmin_hbm, lbl_hbm, w_hbm, out_hbm,
             gbuf, hot0, hot1, gids0, gids1, idx0, idx1,
             cand_v, cand_i, r_v, r_i,
             dvals, idxs, lblv, wvals, p_all, t_ref, t0s, cnt_ref,
             gsem0, gsem1, sem, sem2):
    wid = lax.axis_index("s") * 2 + lax.axis_index("c")
    iota16 = lax.broadcasted_iota(jnp.int32, (16,), 0)
    inf16 = jnp.full((16,), jnp.inf, jnp.float32)
    zero16i = jnp.zeros((16,), jnp.int32)

    def compact():
        cnt = cnt_ref[0]
        cnt16 = jnp.full((16,), cnt, jnp.int32)

        def do_block(blk):
            rr_ = [(r_v[pl.ds(16 * k, 16)], r_i[pl.ds(16 * k, 16)])
                   for k in range(4)]
            sv = []
            si = []
            for k in range(4):
                pos = jnp.full((16,), blk * 64 + 16 * k, jnp.int32) + iota16
                valid = pos < cnt16
                sv.append(jnp.where(valid, cand_v[pl.ds(blk * 64 + 16 * k, 16)],
                                    inf16))
                si.append(cand_i[pl.ds(blk * 64 + 16 * k, 16)])
            rr_ = _merge_keep64(rr_, _sort64(sv, si))
            for k in range(4):
                r_v[pl.ds(16 * k, 16)] = rr_[k][0]
                r_i[pl.ds(16 * k, 16)] = rr_[k][1]

        do_block(0)

        @pl.when(cnt > 64)
        def _():
            do_block(1)

        @pl.when(cnt > 128)
        def _():
            do_block(2)

        t_ref[0] = jnp.max(r_v[pl.ds(48, 16)])
        cnt_ref[0] = 0

    def reset_r():
        for k in range(4):
            r_v[pl.ds(16 * k, 16)] = inf16
            r_i[pl.ds(16 * k, 16)] = zero16i
        cnt_ref[0] = 0

    def group_step(vs, idxvecs):
        m01 = jnp.minimum(vs[0], vs[1])
        m23 = jnp.minimum(vs[2], vs[3])
        m45 = jnp.minimum(vs[4], vs[5])
        m67 = jnp.minimum(vs[6], vs[7])
        mtree = jnp.minimum(jnp.minimum(m01, m23), jnp.minimum(m45, m67))
        mn = jnp.min(mtree)

        @pl.when(mn <= t_ref[0])
        def _():
            @pl.when(cnt_ref[0] > 64)
            def _():
                compact()

            t16 = jnp.full((16,), t_ref[0], jnp.float32)
            for k in range(8):
                mk = vs[k] <= t16
                cnt = cnt_ref[0]
                plsc.store_compressed(cand_v.at[pl.ds(cnt, 16)], vs[k], mask=mk)
                plsc.store_compressed(cand_i.at[pl.ds(cnt, 16)], idxvecs[k],
                                      mask=mk)
                cnt_ref[0] = cnt + jnp.max(plsc.all_reduce_population_count(mk))

    def mini_and_fire(row, hotr, gidsr, idxr, gsem, slot):
        pltpu.sync_copy(gmin_hbm.at[row], gbuf)
        reset_r()
        t_ref[0] = jnp.inf

        def mini_group(g, _):
            base = g * 128
            vs = [gbuf[pl.ds(base + 16 * k, 16)] for k in range(8)]
            ivs = [jnp.full((16,), base + 16 * k, jnp.int32) + iota16
                   for k in range(8)]
            group_step(vs, ivs)
            return 0

        lax.fori_loop(0, NGPAD // 128, mini_group, 0)
        compact()
        row16 = jnp.full((16,), row, jnp.int32)
        for k in range(4):
            gv = r_i[pl.ds(16 * k, 16)]
            gidsr[pl.ds(16 * k, 16)] = gv
            idxr[pl.ds(16 * k, 16)] = gv * B + row16
        t0s[slot] = t_ref[0]
        pltpu.async_copy(sqtab_hbm.at[idxr], hotr, gsem)

    def hot_scan(rr, hotr, gidsr, idxr, gsem, slot):
        pltpu.make_async_copy(sqtab_hbm.at[idxr], hotr, gsem).wait()
        t_ref[0] = t0s[slot]
        reset_r()

        def hot_group(s, _):
            gidv = plsc.load_gather(gidsr, [jnp.full((16,), s, jnp.int32)])
            base16 = gidv * 128
            vs = [hotr[s, pl.ds(16 * k, 16)] for k in range(8)]
            ivs = [base16 + (jnp.full((16,), 16 * k, jnp.int32) + iota16)
                   for k in range(8)]
            group_step(vs, ivs)
            return 0

        lax.fori_loop(0, K, hot_group, 0)
        compact()
        for k in range(4):
            dvals[pl.ds(rr * K + 16 * k, 16)] = r_v[pl.ds(16 * k, 16)]
            idxs[pl.ds(rr * K + 16 * k, 16)] = r_i[pl.ds(16 * k, 16)]

    mini_and_fire(wid * ROWS_PER, hot0, gids0, idx0, gsem0, 0)

    def scan_iter(rr, _):
        row = wid * ROWS_PER + rr
        par = lax.rem(rr, 2)

        @pl.when(par == 0)
        def _():
            @pl.when(rr + 1 < ROWS_PER)
            def _():
                mini_and_fire(row + 1, hot1, gids1, idx1, gsem1, 1)

            hot_scan(rr, hot0, gids0, idx0, gsem0, 0)

        @pl.when(par == 1)
        def _():
            @pl.when(rr + 1 < ROWS_PER)
            def _():
                mini_and_fire(row + 1, hot0, gids0, idx0, gsem0, 0)

            hot_scan(rr, hot1, gids1, idx1, gsem1, 1)

        return 0

    lax.fori_loop(0, ROWS_PER, scan_iter, 0)

    copies = []
    for g in range(ROWS_PER * K // 128):
        sl = pl.ds(g * 128, 128)
        copies.append(pltpu.async_copy(lbl_hbm.at[idxs.at[sl]], lblv.at[sl], sem))
        copies.append(pltpu.async_copy(w_hbm.at[idxs.at[sl]], wvals.at[sl], sem2))
    for cp in copies:
        cp.wait()

    prev_idx = jnp.maximum(iota16 - 1, 0)
    next_idx = jnp.minimum(iota16 + 1, 15)

    def finalize_row(rr, _):
        def zero_body(j, _):
            p_all[pl.ds(rr * CPAD + 16 * j, 16)] = jnp.zeros((16,), jnp.float32)
            return 0

        lax.fori_loop(0, CPAD // 16, zero_body, 0)
        pbase16 = jnp.full((16,), rr * CPAD, jnp.int32)
        for k in range(4):
            off = pl.ds(rr * K + 16 * k, 16)
            e = jnp.exp(wvals[off] - dvals[off] * GC)
            ls, es = plsc.sort_key_val(lblv[off], e)
            cs = plsc.cumsum(es)
            pl_ = jnp.take_along_axis(ls, prev_idx, axis=0,
                                      mode="promise_in_bounds")
            nl_ = jnp.take_along_axis(ls, next_idx, axis=0,
                                      mode="promise_in_bounds")
            is_start = (iota16 == 0) | (ls != pl_)
            is_end = (iota16 == 15) | (ls != nl_)
            pre = jnp.where(iota16 == 0, 0.0,
                            jnp.take_along_axis(cs, prev_idx, axis=0,
                                                mode="promise_in_bounds"))
            plsc.addupdate_scatter(p_all, [pbase16 + ls], cs, mask=is_end)
            plsc.addupdate_scatter(p_all, [pbase16 + ls], -pre, mask=is_start)
        return 0

    lax.fori_loop(0, ROWS_PER, finalize_row, 0)
    pltpu.sync_copy(p_all, out_hbm.at[pl.ds(wid * ROWS_PER * CPAD, ROWS_PER * CPAD)])


_sc_topk = functools.partial(
    pl.kernel,
    out_type=jax.ShapeDtypeStruct((B * CPAD,), jnp.float32),
    mesh=plsc.VectorSubcoreMesh(core_axis_name="c", subcore_axis_name="s"),
    compiler_params=pltpu.CompilerParams(needs_layout_passes=False),
    scratch_types=[
        pltpu.VMEM((NGPAD,), jnp.float32),
        pltpu.VMEM((K, 128), jnp.float32),
        pltpu.VMEM((K, 128), jnp.float32),
        pltpu.VMEM((K,), jnp.int32),
        pltpu.VMEM((K,), jnp.int32),
        pltpu.VMEM((K,), jnp.int32),
        pltpu.VMEM((K,), jnp.int32),
        pltpu.VMEM((CAP,), jnp.float32),
        pltpu.VMEM((CAP,), jnp.int32),
        pltpu.VMEM((K,), jnp.float32),
        pltpu.VMEM((K,), jnp.int32),
        pltpu.VMEM((ROWS_PER * K,), jnp.float32),
        pltpu.VMEM((ROWS_PER * K,), jnp.int32),
        pltpu.VMEM((ROWS_PER * K,), jnp.int32),
        pltpu.VMEM((ROWS_PER * K,), jnp.float32),
        pltpu.VMEM((ROWS_PER * CPAD,), jnp.float32),
        pltpu.SMEM((1,), jnp.float32),
        pltpu.SMEM((2,), jnp.float32),
        pltpu.SMEM((1,), jnp.int32),
        pltpu.SemaphoreType.DMA,
        pltpu.SemaphoreType.DMA,
        pltpu.SemaphoreType.DMA,
        pltpu.SemaphoreType.DMA,
    ],
)(_sc_body)


def _norm_kernel(p_ref, o_ref):
    p = p_ref[...][:, :NCLS]
    p = jnp.where(p == 0.0, 1e-10, p)
    p = p / jnp.sum(p, axis=1, keepdims=True)
    o_ref[...] = jnp.log(p)


def kernel(features, centres, centre_labels, weight):
    sq, gmin = pl.pallas_call(
        _dist_kernel,
        grid=(NMT,),
        in_specs=[
            pl.BlockSpec((B, D), lambda m: (0, 0)),
            pl.BlockSpec((MTILE, D), lambda m: (m, 0)),
        ],
        out_specs=[
            pl.BlockSpec((MTILE // 128, B, 128), lambda m: (m, 0, 0)),
            pl.BlockSpec((1, B, MTILE // 128), lambda m: (m, 0, 0)),
        ],
        out_shape=[
            jax.ShapeDtypeStruct((NG, B, 128), jnp.float32),
            jax.ShapeDtypeStruct((NMT, B, MTILE // 128), jnp.float32),
        ],
    )(features, centres)
    sqtab = sq.reshape(NG * B, 128)
    gmin2 = jnp.concatenate(
        [gmin.transpose(1, 0, 2).reshape(B, NG),
         jnp.full((B, NGPAD - NG), jnp.inf, jnp.float32)], axis=1)
    pflat = _sc_topk(sqtab, gmin2, centre_labels, weight)
    out = pl.pallas_call(
        _norm_kernel,
        out_shape=jax.ShapeDtypeStruct((B, NCLS), jnp.float32),
    )(pflat.reshape(B, CPAD))
    return out
